# baseline (speedup 1.0000x reference)
import functools

import jax
import jax.numpy as jnp
from jax import lax
from jax.experimental import pallas as pl
from jax.experimental.pallas import tpu as pltpu
from jax.experimental.pallas import tpu_sc as plsc

E = 8
TOPK = 2
D = 1024
FF = 1024
T = 256

L = 16
NW = 32
CPW = (T * E) // (NW * L)

_NEG = -3.0e38


def _take(v, idx):
    return lax.gather(
        v, idx[:, None],
        dimension_numbers=lax.GatherDimensionNumbers(
            offset_dims=(), collapsed_slice_dims=(0,), start_index_map=(0,)),
        slice_sizes=(1,),
        mode=lax.GatherScatterMode.PROMISE_IN_BOUNDS)


def _seg_top(val, idx, lane):
    for sh in (4, 2, 1):
        pidx = lax.bitwise_xor(lane, sh)
        v2 = _take(val, pidx)
        i2 = _take(idx, pidx)
        gt = jnp.logical_or(v2 > val,
                            jnp.logical_and(v2 == val, i2 < idx))
        val = jnp.where(gt, v2, val)
        idx = jnp.where(gt, i2, idx)
    return val, idx


def _routing_combine_sc(gating_flat):
    mesh = plsc.VectorSubcoreMesh(core_axis_name="c", subcore_axis_name="s")

    @functools.partial(
        pl.kernel,
        mesh=mesh,
        out_type=jax.ShapeDtypeStruct((T * E,), jnp.float32),
        scratch_types=[
            pltpu.VMEM((CPW * L,), jnp.float32),
            pltpu.VMEM((CPW * L,), jnp.float32),
        ],
    )
    def k(gating_hbm, out_hbm, g_v, o_v):
        wid = lax.axis_index("s") * 2 + lax.axis_index("c")
        base = wid * (CPW * L)
        pltpu.sync_copy(gating_hbm.at[pl.ds(base, CPW * L)], g_v)
        lane = lax.iota(jnp.int32, L)
        eidx = lax.bitwise_and(lane, E - 1)
        for j in range(CPW):
            v = g_v[pl.ds(j * L, L)]
            m1, i1 = _seg_top(v, eidx, lane)
            vm = jnp.where(eidx == i1, jnp.full((L,), _NEG, jnp.float32), v)
            m2, i2 = _seg_top(vm, eidx, lane)
            w1 = 1.0 / (1.0 + jnp.exp(m2 - m1))
            w2 = 1.0 - w1
            zero = jnp.zeros((L,), jnp.float32)
            c = (jnp.where(eidx == i1, w1, zero)
                 + jnp.where(eidx == i2, w2, zero))
            o_v[pl.ds(j * L, L)] = c
        pltpu.sync_copy(o_v, out_hbm.at[pl.ds(base, CPW * L)])

    return k(gating_flat)


def _moe_body(x_ref, combine_ref, gu_ref, down_ref, out_ref):
    e = pl.program_id(0)
    nt = (((1,), (1,)), ((), ()))

    xb = x_ref[...].astype(jnp.bfloat16)
    gate_w = gu_ref[0, :FF].astype(jnp.bfloat16)
    up_w = gu_ref[0, FF:].astype(jnp.bfloat16)
    gg = lax.dot_general(xb, gate_w, nt, preferred_element_type=jnp.float32)
    uu = lax.dot_general(xb, up_w, nt, preferred_element_type=jnp.float32)
    h = gg * jax.nn.sigmoid(gg) * uu
    down_w = down_ref[0].astype(jnp.bfloat16)
    yb = lax.dot_general(h.astype(jnp.bfloat16), down_w, nt,
                         preferred_element_type=jnp.float32)

    cm = combine_ref[...]
    sel = lax.broadcasted_iota(jnp.int32, cm.shape, 1) == e
    col = jnp.sum(jnp.where(sel, cm, 0.0), axis=1, keepdims=True)
    contrib = yb * col

    @pl.when(e == 0)
    def _():
        out_ref[...] = contrib

    @pl.when(e != 0)
    def _():
        out_ref[...] += contrib


@jax.jit
def kernel(x, gating_output, gate_up_proj, down_proj):
    combine = _routing_combine_sc(gating_output.reshape(-1)).reshape(T, E)
    out = pl.pallas_call(
        _moe_body,
        grid=(E,),
        in_specs=[
            pl.BlockSpec((T, D), lambda e: (0, 0)),
            pl.BlockSpec((T, E), lambda e: (0, 0)),
            pl.BlockSpec((1, 2 * FF, D), lambda e: (e, 0, 0)),
            pl.BlockSpec((1, D, FF), lambda e: (e, 0, 0)),
        ],
        out_specs=pl.BlockSpec((T, D), lambda e: (0, 0)),
        out_shape=jax.ShapeDtypeStruct((T, D), jnp.float32),
    )(x, combine, gate_up_proj, down_proj)
    return out


# speedup vs baseline: 1.4768x; 1.4768x over previous
import jax
import jax.numpy as jnp
from jax import lax
from jax.experimental import pallas as pl
from jax.experimental.pallas import tpu as pltpu

E = 8
TOPK = 2
D = 1024
FF = 1024
T = 256


def _combine_from_logits(g):
    iota = lax.broadcasted_iota(jnp.int32, g.shape, 1)
    m1 = jnp.max(g, axis=1, keepdims=True)
    i1 = jnp.min(jnp.where(g == m1, iota, E), axis=1, keepdims=True)
    mask1 = iota == i1
    g_rest = jnp.where(mask1, -jnp.inf, g)
    m2 = jnp.max(g_rest, axis=1, keepdims=True)
    i2 = jnp.min(jnp.where(g_rest == m2, iota, E), axis=1, keepdims=True)
    mask2 = iota == i2
    w1 = jax.nn.sigmoid(m1 - m2)
    w2 = 1.0 - w1
    return jnp.where(mask1, w1, 0.0) + jnp.where(mask2, w2, 0.0)


def _moe_body(x_ref, gating_ref, gu_ref, down_ref, out_ref, combine_ref):
    e = pl.program_id(0)
    nt = (((1,), (1,)), ((), ()))

    @pl.when(e == 0)
    def _():
        combine_ref[...] = _combine_from_logits(gating_ref[...])

    xb = x_ref[...].astype(jnp.bfloat16)
    gate_w = gu_ref[0, :FF].astype(jnp.bfloat16)
    up_w = gu_ref[0, FF:].astype(jnp.bfloat16)
    gg = lax.dot_general(xb, gate_w, nt, preferred_element_type=jnp.float32)
    uu = lax.dot_general(xb, up_w, nt, preferred_element_type=jnp.float32)
    h = gg * jax.nn.sigmoid(gg) * uu
    down_w = down_ref[0].astype(jnp.bfloat16)
    yb = lax.dot_general(h.astype(jnp.bfloat16), down_w, nt,
                         preferred_element_type=jnp.float32)

    cm = combine_ref[...]
    sel = lax.broadcasted_iota(jnp.int32, cm.shape, 1) == e
    col = jnp.sum(jnp.where(sel, cm, 0.0), axis=1, keepdims=True)
    contrib = yb * col

    @pl.when(e == 0)
    def _():
        out_ref[...] = contrib

    @pl.when(e != 0)
    def _():
        out_ref[...] += contrib


@jax.jit
def kernel(x, gating_output, gate_up_proj, down_proj):
    out = pl.pallas_call(
        _moe_body,
        grid=(E,),
        in_specs=[
            pl.BlockSpec((T, D), lambda e: (0, 0)),
            pl.BlockSpec((T, E), lambda e: (0, 0)),
            pl.BlockSpec((1, 2 * FF, D), lambda e: (e, 0, 0)),
            pl.BlockSpec((1, D, FF), lambda e: (e, 0, 0)),
        ],
        out_specs=pl.BlockSpec((T, D), lambda e: (0, 0)),
        out_shape=jax.ShapeDtypeStruct((T, D), jnp.float32),
        scratch_shapes=[
            pltpu.VMEM((T, E), jnp.float32),
        ],
    )(x, gating_output, gate_up_proj, down_proj)
    return out
